# baseline (speedup 1.0000x reference)
import jax
import jax.numpy as jnp
from jax.experimental import pallas as pl
from jax.experimental.pallas import tpu as pltpu

DIM = 2048
N_EXPERTS = 64
TOPK = 2
TOKENS = 16384

BLOCK_T = 2048
HALF = TOKENS // 2


def _top2(scores):
    iota = jax.lax.broadcasted_iota(jnp.int32, scores.shape, 1)
    m1 = jnp.max(scores, axis=1, keepdims=True)
    i1 = jnp.min(jnp.where(scores == m1, iota, N_EXPERTS), axis=1,
                 keepdims=True)
    masked = jnp.where(iota == i1, -1.0, scores)
    m2 = jnp.max(masked, axis=1, keepdims=True)
    i2 = jnp.min(jnp.where(masked == m2, iota, N_EXPERTS), axis=1,
                 keepdims=True)
    denom = jnp.clip(m1 + m2, 1e-12, None)
    w = jnp.concatenate([m1 / denom, m2 / denom], axis=1)
    return w, jnp.concatenate([i1, i2], axis=1)


def _router_kernel(xa_ref, xb_ref, w_ref, owa_ref, oia_ref, owb_ref, oib_ref):
    w = w_ref[...]

    def run(x_ref, ow_ref, oi_ref):
        logits = jax.lax.dot_general(
            x_ref[...], w,
            dimension_numbers=(((1,), (1,)), ((), ())),
            preferred_element_type=jnp.float32,
        )
        gates, idx = _top2(jax.nn.sigmoid(logits))
        ow_ref[...] = gates.astype(ow_ref.dtype)
        oi_ref[...] = idx

    run(xa_ref, owa_ref, oia_ref)
    run(xb_ref, owb_ref, oib_ref)


@jax.jit
def kernel(x, W):
    xa = jax.lax.slice(x, (0, 0), (HALF, DIM))
    xb = jax.lax.slice(x, (HALF, 0), (TOKENS, DIM))
    grid = (HALF // BLOCK_T,)
    owa, oia, owb, oib = pl.pallas_call(
        _router_kernel,
        grid=grid,
        in_specs=[
            pl.BlockSpec((BLOCK_T, DIM), lambda i: (i, 0)),
            pl.BlockSpec((BLOCK_T, DIM), lambda i: (i, 0)),
            pl.BlockSpec((N_EXPERTS, DIM), lambda i: (0, 0)),
        ],
        out_specs=[
            pl.BlockSpec((BLOCK_T, TOPK), lambda i: (i, 0)),
            pl.BlockSpec((BLOCK_T, TOPK), lambda i: (i, 0)),
            pl.BlockSpec((BLOCK_T, TOPK), lambda i: (i, 0)),
            pl.BlockSpec((BLOCK_T, TOPK), lambda i: (i, 0)),
        ],
        out_shape=[
            jax.ShapeDtypeStruct((HALF, TOPK), x.dtype),
            jax.ShapeDtypeStruct((HALF, TOPK), jnp.int32),
            jax.ShapeDtypeStruct((HALF, TOPK), x.dtype),
            jax.ShapeDtypeStruct((HALF, TOPK), jnp.int32),
        ],
        compiler_params=pltpu.CompilerParams(
            dimension_semantics=("parallel",),
        ),
    )(xa, xb, W)
    return (jnp.concatenate([owa, owb], axis=0),
            jnp.concatenate([oia, oib], axis=0))


# speedup vs baseline: 1.8933x; 1.8933x over previous
import jax
import jax.numpy as jnp
from jax.experimental import pallas as pl
from jax.experimental.pallas import tpu as pltpu

DIM = 2048
N_EXPERTS = 64
TOPK = 2
TOKENS = 16384

BLOCK_T = 2048
HALF = TOKENS // 2


def _top2(scores):
    iota = jax.lax.broadcasted_iota(jnp.int32, scores.shape, 1)
    m1 = jnp.max(scores, axis=1, keepdims=True)
    i1 = jnp.min(jnp.where(scores == m1, iota, N_EXPERTS), axis=1,
                 keepdims=True)
    masked = jnp.where(iota == i1, -1.0, scores)
    m2 = jnp.max(masked, axis=1, keepdims=True)
    i2 = jnp.min(jnp.where(masked == m2, iota, N_EXPERTS), axis=1,
                 keepdims=True)
    denom = jnp.clip(m1 + m2, 1e-12, None)
    w = jnp.concatenate([m1 / denom, m2 / denom], axis=1)
    return w, jnp.concatenate([i1, i2], axis=1)


def _router_kernel(xa_ref, xb_ref, w_ref, owa_ref, oia_ref, owb_ref, oib_ref):
    w = w_ref[...]

    def run(x_ref, ow_ref, oi_ref):
        logits = jax.lax.dot_general(
            x_ref[...], w,
            dimension_numbers=(((1,), (1,)), ((), ())),
            preferred_element_type=jnp.float32,
        )
        gates, idx = _top2(jax.nn.sigmoid(logits))
        ow_ref[...] = gates.astype(ow_ref.dtype)
        oi_ref[...] = idx

    run(xa_ref, owa_ref, oia_ref)
    run(xb_ref, owb_ref, oib_ref)


@jax.jit
def kernel(x, W):
    n_steps = HALF // BLOCK_T
    grid = (n_steps,)
    owa, oia, owb, oib = pl.pallas_call(
        _router_kernel,
        grid=grid,
        in_specs=[
            pl.BlockSpec((BLOCK_T, DIM), lambda i: (i, 0)),
            pl.BlockSpec((BLOCK_T, DIM), lambda i: (i + HALF // BLOCK_T, 0)),
            pl.BlockSpec((N_EXPERTS, DIM), lambda i: (0, 0)),
        ],
        out_specs=[
            pl.BlockSpec((BLOCK_T, TOPK), lambda i: (i, 0)),
            pl.BlockSpec((BLOCK_T, TOPK), lambda i: (i, 0)),
            pl.BlockSpec((BLOCK_T, TOPK), lambda i: (i, 0)),
            pl.BlockSpec((BLOCK_T, TOPK), lambda i: (i, 0)),
        ],
        out_shape=[
            jax.ShapeDtypeStruct((HALF, TOPK), x.dtype),
            jax.ShapeDtypeStruct((HALF, TOPK), jnp.int32),
            jax.ShapeDtypeStruct((HALF, TOPK), x.dtype),
            jax.ShapeDtypeStruct((HALF, TOPK), jnp.int32),
        ],
        compiler_params=pltpu.CompilerParams(
            dimension_semantics=("parallel",),
        ),
    )(x, x, W)
    return (jnp.concatenate([owa, owb], axis=0),
            jnp.concatenate([oia, oib], axis=0))


# speedup vs baseline: 3.1205x; 1.6481x over previous
import jax
import jax.numpy as jnp
from jax.experimental import pallas as pl
from jax.experimental.pallas import tpu as pltpu

DIM = 2048
N_EXPERTS = 64
TOPK = 2
TOKENS = 16384

BLOCK_T = 2048


def _router_kernel(x_ref, w_ref, out_w_ref, out_i_ref):
    logits = jax.lax.dot_general(
        w_ref[...], x_ref[...],
        dimension_numbers=(((1,), (1,)), ((), ())),
        preferred_element_type=jnp.float32,
    )
    scores = jax.nn.sigmoid(logits)

    iota = jax.lax.broadcasted_iota(jnp.int32, scores.shape, 0)
    m1 = jnp.max(scores, axis=0, keepdims=True)
    i1 = jnp.min(jnp.where(scores == m1, iota, N_EXPERTS), axis=0,
                 keepdims=True)
    masked = jnp.where(iota == i1, -1.0, scores)
    m2 = jnp.max(masked, axis=0, keepdims=True)
    i2 = jnp.min(jnp.where(masked == m2, iota, N_EXPERTS), axis=0,
                 keepdims=True)

    denom = jnp.clip(m1 + m2, 1e-12, None)
    w1 = m1 / denom
    w2 = m2 / denom
    out_w_ref[...] = jnp.concatenate([w1, w2], axis=0).astype(out_w_ref.dtype)
    out_i_ref[...] = jnp.concatenate([i1, i2], axis=0)


@jax.jit
def kernel(x, W):
    grid = (TOKENS // BLOCK_T,)
    out_w, out_i = pl.pallas_call(
        _router_kernel,
        grid=grid,
        in_specs=[
            pl.BlockSpec((BLOCK_T, DIM), lambda i: (i, 0)),
            pl.BlockSpec((N_EXPERTS, DIM), lambda i: (0, 0)),
        ],
        out_specs=[
            pl.BlockSpec((TOPK, BLOCK_T), lambda i: (0, i)),
            pl.BlockSpec((TOPK, BLOCK_T), lambda i: (0, i)),
        ],
        out_shape=[
            jax.ShapeDtypeStruct((TOPK, TOKENS), x.dtype),
            jax.ShapeDtypeStruct((TOPK, TOKENS), jnp.int32),
        ],
        compiler_params=pltpu.CompilerParams(
            dimension_semantics=("parallel",),
        ),
    )(x, W)
    return (out_w.T, out_i.T)
